# baseline (speedup 1.0000x reference)
import functools

import jax
import jax.numpy as jnp
from jax import lax
from jax.experimental import pallas as pl
from jax.experimental.pallas import tpu as pltpu
from jax.experimental.pallas import tpu_sc as plsc

D = 64
NC, NS = 2, 16
NW = NC * NS
IDX_PER_XFER = 128
J = 50


def _make_gather(total):
    b_per_w = total // NW
    assert b_per_w == J * IDX_PER_XFER
    mesh = plsc.VectorSubcoreMesh(core_axis_name="c", subcore_axis_name="s")

    @functools.partial(
        pl.kernel,
        mesh=mesh,
        compiler_params=pltpu.CompilerParams(use_tc_tiling_on_sc=False),
        out_type=jax.ShapeDtypeStruct((total, D), jnp.float32),
        scratch_types=[
            pltpu.VMEM((J, IDX_PER_XFER), jnp.int32),
            pltpu.VMEM((IDX_PER_XFER, D), jnp.float32),
            pltpu.SemaphoreType.DMA,
        ],
    )
    def gather_kernel(words_hbm, table_hbm, out_hbm, idx_v, rows_v, gsem):
        wid = lax.axis_index("s") * NC + lax.axis_index("c")
        base = wid * b_per_w
        pltpu.sync_copy(words_hbm.at[wid], idx_v)

        def body(j, carry):
            pltpu.async_copy(table_hbm.at[idx_v.at[j]], rows_v, gsem).wait()
            pltpu.sync_copy(
                rows_v, out_hbm.at[pl.ds(base + j * IDX_PER_XFER, IDX_PER_XFER)]
            )
            return carry

        lax.fori_loop(0, J, body, 0)

    return gather_kernel


def kernel(words, table):
    b, s = words.shape
    total = b * s
    words3 = words.reshape(NW, J, IDX_PER_XFER)
    out = _make_gather(total)(words3, table)
    return out.reshape(b, s, D)


# speedup vs baseline: 1.0468x; 1.0468x over previous
import functools

import jax
import jax.numpy as jnp
from jax import lax
from jax.experimental import pallas as pl
from jax.experimental.pallas import tpu as pltpu
from jax.experimental.pallas import tpu_sc as plsc

D = 64
NC, NS = 2, 16
NW = NC * NS
IDX_PER_XFER = 128
G = 5
NG = 10
GROUP_ROWS = G * IDX_PER_XFER
J = G * NG


def _make_gather(total):
    b_per_w = total // NW
    assert b_per_w == J * IDX_PER_XFER
    mesh = plsc.VectorSubcoreMesh(core_axis_name="c", subcore_axis_name="s")

    @functools.partial(
        pl.kernel,
        mesh=mesh,
        compiler_params=pltpu.CompilerParams(use_tc_tiling_on_sc=False),
        out_type=jax.ShapeDtypeStruct((total, D), jnp.float32),
        scratch_types=[
            pltpu.VMEM((J, IDX_PER_XFER), jnp.int32),
            pltpu.VMEM((GROUP_ROWS, D), jnp.float32),
            pltpu.VMEM((GROUP_ROWS, D), jnp.float32),
            pltpu.SemaphoreType.DMA,
            pltpu.SemaphoreType.DMA,
            pltpu.SemaphoreType.DMA,
            pltpu.SemaphoreType.DMA,
        ],
    )
    def gather_kernel(words_hbm, table_hbm, out_hbm,
                      idx_v, rows0, rows1, gsem0, gsem1, osem0, osem1):
        wid = lax.axis_index("s") * NC + lax.axis_index("c")
        base = wid * b_per_w
        pltpu.sync_copy(words_hbm.at[wid], idx_v)

        def fire_group(g, rows, gsem):
            for i in range(G):
                pltpu.async_copy(
                    table_hbm.at[idx_v.at[g * G + i]],
                    rows.at[pl.ds(i * IDX_PER_XFER, IDX_PER_XFER)],
                    gsem,
                )

        def drain_group(rows, gsem):
            for i in range(G):
                pltpu.make_async_copy(
                    table_hbm.at[idx_v.at[0]],
                    rows.at[pl.ds(i * IDX_PER_XFER, IDX_PER_XFER)],
                    gsem,
                ).wait()

        def out_start(g, rows, osem):
            pltpu.async_copy(
                rows, out_hbm.at[pl.ds(base + g * GROUP_ROWS, GROUP_ROWS)], osem
            )

        def out_wait(rows, osem):
            pltpu.make_async_copy(
                rows, out_hbm.at[pl.ds(base, GROUP_ROWS)], osem
            ).wait()

        fire_group(0, rows0, gsem0)
        fire_group(1, rows1, gsem1)

        def body(k, carry):
            g0 = 2 * k
            drain_group(rows0, gsem0)
            out_start(g0, rows0, osem0)
            out_wait(rows0, osem0)
            fire_group(g0 + 2, rows0, gsem0)
            drain_group(rows1, gsem1)
            out_start(g0 + 1, rows1, osem1)
            out_wait(rows1, osem1)
            fire_group(g0 + 3, rows1, gsem1)
            return carry

        lax.fori_loop(0, NG // 2 - 1, body, 0)

        g_last = NG - 2
        drain_group(rows0, gsem0)
        out_start(g_last, rows0, osem0)
        drain_group(rows1, gsem1)
        out_start(g_last + 1, rows1, osem1)
        out_wait(rows0, osem0)
        out_wait(rows1, osem1)

    return gather_kernel


def kernel(words, table):
    b, s = words.shape
    total = b * s
    words3 = words.reshape(NW, J, IDX_PER_XFER)
    out = _make_gather(total)(words3, table)
    return out.reshape(b, s, D)
